# baseline (speedup 1.0000x reference)
import functools

import jax
import jax.numpy as jnp
from jax import lax
from jax.experimental import pallas as pl
from jax.experimental.pallas import tpu as pltpu
from jax.experimental.pallas import tpu_sc as plsc

_NUM_CORES = 2
_NUM_SUBCORES = 16
_NW = _NUM_CORES * _NUM_SUBCORES
_CHUNK = 128
_NBUF = 2


@functools.lru_cache(maxsize=None)
def _make_gather(b0: int, b1: int, d: int):
    b = b0 * b1
    assert b % (_NW * _CHUNK) == 0 and b1 % _NW == 0 or _NW % b1 == 0
    b_per_w = b // _NW
    n_chunks = b_per_w // _CHUNK
    w_per_row = _NW // b0

    mesh = plsc.VectorSubcoreMesh(core_axis_name="c", subcore_axis_name="s")
    scratch = [pltpu.VMEM((b_per_w,), jnp.int32)]
    scratch += [pltpu.VMEM((_CHUNK, d), jnp.float32) for _ in range(_NBUF)]
    scratch += [pltpu.SemaphoreType.DMA for _ in range(2 * _NBUF)]

    @functools.partial(
        pl.kernel,
        mesh=mesh,
        out_type=jax.ShapeDtypeStruct((b, d), jnp.float32),
        scratch_types=scratch,
    )
    def gather_kernel(idx_hbm, table_hbm, out_hbm, idx_v, *rest):
        bufs = rest[:_NBUF]
        gsems = rest[_NBUF:2 * _NBUF]
        ssems = rest[2 * _NBUF:]
        wid = lax.axis_index("s") * _NUM_CORES + lax.axis_index("c")
        base = wid * b_per_w
        row = wid // w_per_row
        col = (wid % w_per_row) * b_per_w
        pltpu.sync_copy(idx_hbm.at[row, pl.ds(col, b_per_w)], idx_v)

        def gather_copy(c, b):
            return pltpu.make_async_copy(
                table_hbm.at[idx_v.at[pl.ds(c * _CHUNK, _CHUNK)]],
                bufs[b], gsems[b])

        def scatter_copy(c, b):
            return pltpu.make_async_copy(
                bufs[b],
                out_hbm.at[pl.ds(base + c * _CHUNK, _CHUNK)],
                ssems[b])

        for b in range(_NBUF):
            gather_copy(b, b).start()

        n_groups = n_chunks // _NBUF

        def group(g, carry):
            c0 = g * _NBUF
            for b in range(_NBUF):
                gather_copy(c0 + b, b).wait()
                scatter_copy(c0 + b, b).start()

            @pl.when(g < n_groups - 1)
            def _():
                for b in range(_NBUF):
                    scatter_copy(c0 + b, b).wait()
                    gather_copy(c0 + _NBUF + b, b).start()
            return carry

        lax.fori_loop(0, n_groups, group, 0, unroll=False)
        for b in range(_NBUF):
            scatter_copy(n_chunks - _NBUF + b, b).wait()

    return gather_kernel


def kernel(embed_id, weight):
    b0, b1 = embed_id.shape
    d = weight.shape[1]
    idx = jnp.asarray(embed_id, jnp.int32)
    out = _make_gather(b0, b1, d)(idx, weight)
    return out.reshape(b0, b1, d)


# speedup vs baseline: 1.0133x; 1.0133x over previous
import functools

import jax
import jax.numpy as jnp
from jax import lax
from jax.experimental import pallas as pl
from jax.experimental.pallas import tpu as pltpu
from jax.experimental.pallas import tpu_sc as plsc

_NUM_CORES = 2
_NUM_SUBCORES = 16
_NW = _NUM_CORES * _NUM_SUBCORES
_CHUNK = 128
_NBUF = 3


@functools.lru_cache(maxsize=None)
def _make_gather(b0: int, b1: int, d: int):
    b = b0 * b1
    assert b % (_NW * _CHUNK) == 0 and _NW % b0 == 0
    b_per_w = b // _NW
    n_chunks = b_per_w // _CHUNK
    w_per_row = _NW // b0

    mesh = plsc.VectorSubcoreMesh(core_axis_name="c", subcore_axis_name="s")
    scratch = [pltpu.VMEM((b_per_w,), jnp.int32)]
    scratch += [pltpu.VMEM((_CHUNK, d), jnp.float32) for _ in range(_NBUF)]
    scratch += [pltpu.SemaphoreType.DMA for _ in range(2 * _NBUF + 1)]

    @functools.partial(
        pl.kernel,
        mesh=mesh,
        out_type=jax.ShapeDtypeStruct((b, d), jnp.float32),
        scratch_types=scratch,
    )
    def gather_kernel(idx_hbm, table_hbm, out_hbm, idx_v, *rest):
        bufs = rest[:_NBUF]
        gsems = rest[_NBUF:2 * _NBUF]
        ssems = rest[2 * _NBUF:3 * _NBUF]
        isem = rest[3 * _NBUF]
        wid = lax.axis_index("s") * _NUM_CORES + lax.axis_index("c")
        base = wid * b_per_w
        row = wid // w_per_row
        col = (wid % w_per_row) * b_per_w

        def idx_copy(c):
            return pltpu.make_async_copy(
                idx_hbm.at[row, pl.ds(col + c * _CHUNK, _CHUNK)],
                idx_v.at[pl.ds(c * _CHUNK, _CHUNK)], isem)

        def gather_copy(c):
            return pltpu.make_async_copy(
                table_hbm.at[idx_v.at[pl.ds(c * _CHUNK, _CHUNK)]],
                bufs[c % _NBUF], gsems[c % _NBUF])

        def scatter_copy(c):
            return pltpu.make_async_copy(
                bufs[c % _NBUF],
                out_hbm.at[pl.ds(base + c * _CHUNK, _CHUNK)],
                ssems[c % _NBUF])

        for c in range(n_chunks):
            idx_copy(c).start()

        gath = {}
        scat = {}
        for c in range(min(_NBUF, n_chunks)):
            idx_copy(c).wait()
            gath[c] = gather_copy(c)
            gath[c].start()
        for c in range(n_chunks):
            gath.pop(c).wait()
            scat[c] = scatter_copy(c)
            scat[c].start()
            nxt = c + _NBUF
            if nxt < n_chunks:
                scat.pop(c).wait()
                idx_copy(nxt).wait()
                gath[nxt] = gather_copy(nxt)
                gath[nxt].start()
        for c in sorted(scat):
            scat[c].wait()

    return gather_kernel


def kernel(embed_id, weight):
    b0, b1 = embed_id.shape
    d = weight.shape[1]
    idx = jnp.asarray(embed_id, jnp.int32)
    out = _make_gather(b0, b1, d)(idx, weight)
    return out.reshape(b0, b1, d)


# speedup vs baseline: 1.0146x; 1.0014x over previous
import functools

import jax
import jax.numpy as jnp
from jax import lax
from jax.experimental import pallas as pl
from jax.experimental.pallas import tpu as pltpu
from jax.experimental.pallas import tpu_sc as plsc

_NUM_CORES = 2
_NUM_SUBCORES = 16
_NW = _NUM_CORES * _NUM_SUBCORES
_CHUNK = 128
_NBUF = 3


@functools.lru_cache(maxsize=None)
def _make_gather(b0: int, b1: int, d: int):
    b = b0 * b1
    assert b % (_NW * _CHUNK) == 0 and _NW % b0 == 0
    b_per_w = b // _NW
    n_chunks = b_per_w // _CHUNK
    w_per_row = _NW // b0

    mesh = plsc.VectorSubcoreMesh(core_axis_name="c", subcore_axis_name="s")
    scratch = [pltpu.VMEM((_CHUNK,), jnp.int32) for _ in range(n_chunks)]
    scratch += [pltpu.VMEM((_CHUNK, d), jnp.float32) for _ in range(_NBUF)]
    scratch += [pltpu.SemaphoreType.DMA for _ in range(2 * _NBUF + 1)]

    @functools.partial(
        pl.kernel,
        mesh=mesh,
        out_type=jax.ShapeDtypeStruct((b, d), jnp.float32),
        scratch_types=scratch,
    )
    def gather_kernel(idx_hbm, table_hbm, out_hbm, *rest):
        idx_vs = rest[:n_chunks]
        bufs = rest[n_chunks:n_chunks + _NBUF]
        gsems = rest[n_chunks + _NBUF:n_chunks + 2 * _NBUF]
        ssems = rest[n_chunks + 2 * _NBUF:n_chunks + 3 * _NBUF]
        isem = rest[n_chunks + 3 * _NBUF]
        wid = lax.axis_index("s") * _NUM_CORES + lax.axis_index("c")
        base = wid * b_per_w
        row = wid // w_per_row
        col = (wid % w_per_row) * b_per_w

        def idx_copy(c):
            return pltpu.make_async_copy(
                idx_hbm.at[row, pl.ds(col + c * _CHUNK, _CHUNK)],
                idx_vs[c], isem)

        def gather_copy(c):
            return pltpu.make_async_copy(
                table_hbm.at[idx_vs[c]],
                bufs[c % _NBUF], gsems[c % _NBUF])

        def scatter_copy(c):
            return pltpu.make_async_copy(
                bufs[c % _NBUF],
                out_hbm.at[pl.ds(base + c * _CHUNK, _CHUNK)],
                ssems[c % _NBUF])

        for c in range(n_chunks):
            idx_copy(c).start()

        gath = {}
        scat = {}
        for c in range(min(_NBUF, n_chunks)):
            idx_copy(c).wait()
            gath[c] = gather_copy(c)
            gath[c].start()
        for c in range(n_chunks):
            gath.pop(c).wait()
            scat[c] = scatter_copy(c)
            scat[c].start()
            nxt = c + _NBUF
            if nxt < n_chunks:
                scat.pop(c).wait()
                idx_copy(nxt).wait()
                gath[nxt] = gather_copy(nxt)
                gath[nxt].start()
        for c in sorted(scat):
            scat[c].wait()

    return gather_kernel


def kernel(embed_id, weight):
    b0, b1 = embed_id.shape
    d = weight.shape[1]
    idx = jnp.asarray(embed_id, jnp.int32)
    out = _make_gather(b0, b1, d)(idx, weight)
    return out.reshape(b0, b1, d)


# speedup vs baseline: 1.0178x; 1.0031x over previous
import functools

import jax
import jax.numpy as jnp
from jax import lax
from jax.experimental import pallas as pl
from jax.experimental.pallas import tpu as pltpu
from jax.experimental.pallas import tpu_sc as plsc

_NUM_CORES = 2
_NUM_SUBCORES = 16
_NW = _NUM_CORES * _NUM_SUBCORES
_CHUNKS = (64, 128, 128, 128, 64)
_BUF = (0, 1, 2, 3, 0)
_NBUF = 4


@functools.lru_cache(maxsize=None)
def _make_gather(b0: int, b1: int, d: int):
    b = b0 * b1
    b_per_w = b // _NW
    assert b % _NW == 0 and _NW % b0 == 0
    assert sum(_CHUNKS) == b_per_w
    n_chunks = len(_CHUNKS)
    starts = [sum(_CHUNKS[:c]) for c in range(n_chunks)]
    w_per_row = _NW // b0

    mesh = plsc.VectorSubcoreMesh(core_axis_name="c", subcore_axis_name="s")
    scratch = [pltpu.VMEM((b_per_w,), jnp.int32)]
    scratch += [pltpu.VMEM((_CHUNKS[bi], d), jnp.float32)
                for bi in range(_NBUF)]
    scratch += [pltpu.SemaphoreType.DMA for _ in range(2 * _NBUF + 1)]

    @functools.partial(
        pl.kernel,
        mesh=mesh,
        out_type=jax.ShapeDtypeStruct((b, d), jnp.float32),
        scratch_types=scratch,
    )
    def gather_kernel(idx_hbm, table_hbm, out_hbm, idx_v, *rest):
        bufs = rest[:_NBUF]
        gsems = rest[_NBUF:2 * _NBUF]
        ssems = rest[2 * _NBUF:3 * _NBUF]
        isem = rest[3 * _NBUF]
        wid = lax.axis_index("s") * _NUM_CORES + lax.axis_index("c")
        base = wid * b_per_w
        row = wid // w_per_row
        col = (wid % w_per_row) * b_per_w

        def idx_copy():
            return pltpu.make_async_copy(
                idx_hbm.at[row, pl.ds(col, b_per_w)], idx_v, isem)

        idx_copy().start()

        def gather_copy(c):
            bi = _BUF[c]
            return pltpu.make_async_copy(
                table_hbm.at[idx_v.at[pl.ds(starts[c], _CHUNKS[c])]],
                bufs[bi], gsems[bi])

        def scatter_copy(c):
            bi = _BUF[c]
            return pltpu.make_async_copy(
                bufs[bi],
                out_hbm.at[pl.ds(base + starts[c], _CHUNKS[c])],
                ssems[bi])

        idx_copy().wait()

        gath = {}
        scat = {}
        for c in range(min(_NBUF, n_chunks)):
            gath[c] = gather_copy(c)
            gath[c].start()
        for c in range(n_chunks):
            gath.pop(c).wait()
            scat[c] = scatter_copy(c)
            scat[c].start()
            nxt = c + _NBUF
            if nxt < n_chunks:
                scat.pop(c).wait()
                gath[nxt] = gather_copy(nxt)
                gath[nxt].start()
        for c in sorted(scat):
            scat[c].wait()

    return gather_kernel


def kernel(embed_id, weight):
    b0, b1 = embed_id.shape
    d = weight.shape[1]
    idx = jnp.asarray(embed_id, jnp.int32)
    out = _make_gather(b0, b1, d)(idx, weight)
    return out.reshape(b0, b1, d)


# speedup vs baseline: 1.0291x; 1.0111x over previous
import functools

import jax
import jax.numpy as jnp
from jax import lax
from jax.experimental import pallas as pl
from jax.experimental.pallas import tpu as pltpu
from jax.experimental.pallas import tpu_sc as plsc

_NUM_CORES = 2
_NUM_SUBCORES = 16
_NW = _NUM_CORES * _NUM_SUBCORES
_CHUNKS = (128, 128, 128, 64, 64)
_BUF = (0, 1, 2, 3, 3)
_NBUF = 4


@functools.lru_cache(maxsize=None)
def _make_gather(b0: int, b1: int, d: int):
    b = b0 * b1
    b_per_w = b // _NW
    assert b % _NW == 0 and _NW % b0 == 0
    assert sum(_CHUNKS) == b_per_w
    n_chunks = len(_CHUNKS)
    starts = [sum(_CHUNKS[:c]) for c in range(n_chunks)]
    w_per_row = _NW // b0
    row_bytes = d * 4

    mesh = plsc.VectorSubcoreMesh(core_axis_name="c", subcore_axis_name="s")
    scratch = [pltpu.VMEM((b_per_w,), jnp.int32)]
    scratch += [pltpu.VMEM((_CHUNKS[bi], d), jnp.float32)
                for bi in range(_NBUF)]
    scratch += [pltpu.SemaphoreType.DMA for _ in range(4)]

    @functools.partial(
        pl.kernel,
        mesh=mesh,
        out_type=jax.ShapeDtypeStruct((b, d), jnp.float32),
        scratch_types=scratch,
    )
    def gather_kernel(idx_hbm, table_hbm, out_hbm, idx_v,
                      buf0, buf1, buf2, buf3, isem, gsem, ssem, rsem):
        bufs = (buf0, buf1, buf2, buf3)
        wid = lax.axis_index("s") * _NUM_CORES + lax.axis_index("c")
        base = wid * b_per_w
        row = wid // w_per_row
        col = (wid % w_per_row) * b_per_w

        def idx_copy():
            return pltpu.make_async_copy(
                idx_hbm.at[row, pl.ds(col, b_per_w)], idx_v, isem)

        def gather_copy(c):
            return pltpu.make_async_copy(
                table_hbm.at[idx_v.at[pl.ds(starts[c], _CHUNKS[c])]],
                bufs[_BUF[c]], gsem)

        def scatter_copy(c, sem):
            return pltpu.make_async_copy(
                bufs[_BUF[c]],
                out_hbm.at[pl.ds(base + starts[c], _CHUNKS[c])], sem)

        idx_copy().start()
        idx_copy().wait()

        for c in range(4):
            gather_copy(c).start()
        for c in range(4):
            gather_copy(c).wait()

        for c in range(3):
            scatter_copy(c, ssem).start()
        scatter_copy(3, rsem).start()
        scatter_copy(3, rsem).wait()

        gather_copy(4).start()
        gather_copy(4).wait()
        scatter_copy(4, ssem).start()

        for c in range(3):
            scatter_copy(c, ssem).wait()
        scatter_copy(4, ssem).wait()

    return gather_kernel


def kernel(embed_id, weight):
    b0, b1 = embed_id.shape
    d = weight.shape[1]
    idx = jnp.asarray(embed_id, jnp.int32)
    out = _make_gather(b0, b1, d)(idx, weight)
    return out.reshape(b0, b1, d)
